# baseline (speedup 1.0000x reference)
import jax
import jax.numpy as jnp
from jax import lax
from jax.experimental import pallas as pl
from jax.experimental.pallas import tpu as pltpu
from jax.experimental.pallas import tpu_sc as plsc

N = 10000
V = 8000
E = 320000
IN = 128
HEADS = 8
OUT = 16
HC = HEADS * OUT
HH = HC // 2
NEG = 0.2

NC, NS = 2, 16
NW = NC * NS
B = 128
CPT = 158
NCHUNK = CPT * NS
EPAD = NCHUNK * B
NP = N + 8

R_BIG = 640
R_SMALL = N - 15 * R_BIG
VPAD = 8192
VPW = VPAD // NW

_f32 = jnp.float32


def _proj(x, w, as_ref, ad_ref):
    h = lax.dot_general(x, w, (((1,), (1,)), ((), ())),
                        preferred_element_type=_f32)
    s = lax.dot_general(h, as_ref, (((1,), (0,)), ((), ())),
                        preferred_element_type=_f32)
    d = lax.dot_general(h, ad_ref, (((1,), (0,)), ((), ())),
                        preferred_element_type=_f32)
    return h, s, d


def _proj_body(x_ref, w_ref, as_ref, ad_ref, hlo_ref, hhi_ref, s_ref, d_ref):
    h, s, d = _proj(x_ref[...], w_ref[...], as_ref[...], ad_ref[...])
    hlo_ref[...] = h[:, :HH]
    hhi_ref[...] = h[:, HH:]
    s_ref[...] = s
    d_ref[...] = d


def _merge_x(acc_ref, den_ref, b16_ref, bias_ref):
    acc = jnp.concatenate([acc_ref[0], acc_ref[1]], axis=1)
    den = den_ref[0]
    denb = lax.dot_general(den, b16_ref[...], (((1,), (0,)), ((), ())),
                           preferred_element_type=_f32)
    return acc / (denb + 1e-16) + bias_ref[...]


def _norm_proj_body(acc_ref, den_ref, b16_ref, bias_ref, w_ref, as_ref,
                    ad_ref, hlo_ref, hhi_ref, s_ref, d_ref):
    x = _merge_x(acc_ref, den_ref, b16_ref, bias_ref)
    h, s, d = _proj(x, w_ref[...], as_ref[...], ad_ref[...])
    hlo_ref[...] = h[:, :HH]
    hhi_ref[...] = h[:, HH:]
    s_ref[...] = s
    d_ref[...] = d


def _norm_body(acc_ref, den_ref, b16_ref, bias_ref, x_ref):
    x_ref[...] = _merge_x(acc_ref, den_ref, b16_ref, bias_ref)


_R = 1000


def _tc_proj(x, w, asel, adsel):
    return pl.pallas_call(
        _proj_body,
        grid=(N // _R,),
        in_specs=[
            pl.BlockSpec((_R, IN), lambda i: (i, 0)),
            pl.BlockSpec((HC, IN), lambda i: (0, 0)),
            pl.BlockSpec((HC, 16), lambda i: (0, 0)),
            pl.BlockSpec((HC, 16), lambda i: (0, 0)),
        ],
        out_specs=[
            pl.BlockSpec((_R, HH), lambda i: (i, 0)),
            pl.BlockSpec((_R, HH), lambda i: (i, 0)),
            pl.BlockSpec((_R, 16), lambda i: (i, 0)),
            pl.BlockSpec((_R, 16), lambda i: (i, 0)),
        ],
        out_shape=[
            jax.ShapeDtypeStruct((N, HH), _f32),
            jax.ShapeDtypeStruct((N, HH), _f32),
            jax.ShapeDtypeStruct((N, 16), _f32),
            jax.ShapeDtypeStruct((N, 16), _f32),
        ],
    )(x, w, asel, adsel)


def _tc_norm_proj(accp, denp, b16, bias, w, asel, adsel):
    return pl.pallas_call(
        _norm_proj_body,
        grid=(N // _R,),
        in_specs=[
            pl.BlockSpec((NC, _R, HH), lambda i: (0, i, 0)),
            pl.BlockSpec((NC, _R, 16), lambda i: (0, i, 0)),
            pl.BlockSpec((16, HC), lambda i: (0, 0)),
            pl.BlockSpec((HC,), lambda i: (0,)),
            pl.BlockSpec((HC, IN), lambda i: (0, 0)),
            pl.BlockSpec((HC, 16), lambda i: (0, 0)),
            pl.BlockSpec((HC, 16), lambda i: (0, 0)),
        ],
        out_specs=[
            pl.BlockSpec((_R, HH), lambda i: (i, 0)),
            pl.BlockSpec((_R, HH), lambda i: (i, 0)),
            pl.BlockSpec((_R, 16), lambda i: (i, 0)),
            pl.BlockSpec((_R, 16), lambda i: (i, 0)),
        ],
        out_shape=[
            jax.ShapeDtypeStruct((N, HH), _f32),
            jax.ShapeDtypeStruct((N, HH), _f32),
            jax.ShapeDtypeStruct((N, 16), _f32),
            jax.ShapeDtypeStruct((N, 16), _f32),
        ],
    )(accp, denp, b16, bias, w, asel, adsel)


def _tc_norm(accp, denp, b16, bias):
    return pl.pallas_call(
        _norm_body,
        grid=(N // _R,),
        in_specs=[
            pl.BlockSpec((NC, _R, HH), lambda i: (0, i, 0)),
            pl.BlockSpec((NC, _R, 16), lambda i: (0, i, 0)),
            pl.BlockSpec((16, HC), lambda i: (0, 0)),
            pl.BlockSpec((HC,), lambda i: (0,)),
        ],
        out_specs=pl.BlockSpec((_R, HC), lambda i: (i, 0)),
        out_shape=jax.ShapeDtypeStruct((N, HC), _f32),
    )(accp, denp, b16, bias)


_MESH = dict(core_axis_name="c", subcore_axis_name="s", num_cores=NC,
             num_subcores=NS)

NSLOT = 3
PF = 2
LSTEPS = CPT - PF


def _edge_body(src_hbm, dst_hbm, s_hbm, d_hbm, hlo_hbm, hhi_hbm,
               accp_hbm, denp_hbm,
               sidx0, sidx1, sidx2,
               didx0, didx1, didx2,
               srow0, srow1, srow2,
               drow0, drow1, drow2,
               rows0, rows1, rows2,
               h_st, acc_sh, den_sh,
               g0, g1, g2, g3, g4, g5, g6, g7, g8):
    cid = lax.axis_index("c")
    sid = lax.axis_index("s")
    gsa = (g0, g1, g2)
    gsb = (g3, g4, g5)
    gsc = (g6, g7, g8)
    sidxs = (sidx0, sidx1, sidx2)
    didxs = (didx0, didx1, didx2)
    srows = (srow0, srow1, srow2)
    drows = (drow0, drow1, drow2)
    rowss = (rows0, rows1, rows2)

    zero16 = jnp.zeros((16,), _f32)

    def _issue(c, par):
        base = c * B
        pltpu.sync_copy(src_hbm.at[pl.ds(base, B)], sidxs[par])
        pltpu.sync_copy(dst_hbm.at[pl.ds(base, B)], didxs[par])
        pltpu.async_copy(s_hbm.at[sidxs[par]], srows[par], gsa[par])
        pltpu.async_copy(d_hbm.at[didxs[par]], drows[par], gsb[par])
        pltpu.async_copy(h_st.at[sidxs[par]], rowss[par], gsc[par])

    def _process(par):
        srow, drow, rows = srows[par], drows[par], rowss[par]
        pltpu.make_async_copy(s_hbm.at[sidxs[par]], srow, gsa[par]).wait()
        pltpu.make_async_copy(d_hbm.at[didxs[par]], drow, gsb[par]).wait()
        pltpu.make_async_copy(h_st.at[sidxs[par]], rows, gsc[par]).wait()

        def _compute(hbase):
            @plsc.parallel_loop(0, B, unroll=2)
            def _(b):
                a = srow[b, :] + drow[b, :]
                a = jnp.where(a > 0, a, a * NEG)
                pv = jnp.exp(a)
                srow[b, :] = pv
                for j in range(HEADS // 2):
                    sc = jnp.full((16,), pv[hbase + j], _f32)
                    rows[b, pl.ds(16 * j, 16)] = (
                        rows[b, pl.ds(16 * j, 16)] * sc)

        @pl.when(cid == 0)
        def _():
            _compute(0)

        @pl.when(cid == 1)
        def _():
            _compute(HEADS // 2)

        pltpu.sync_copy(srow, den_sh.at[didxs[par]], add=True)
        pltpu.sync_copy(rows, acc_sh.at[didxs[par]], add=True)

    r0 = sid * R_BIG
    n_z = jnp.where(sid < 15, R_BIG // 80, R_SMALL // 80)

    @pl.loop(0, 80)
    def _(b):
        srows[0][b, :] = zero16
        for j in range(HEADS // 2):
            rowss[0][b, pl.ds(16 * j, 16)] = zero16

    @pl.when((sid < 15) & (cid == 0))
    def _():
        pltpu.sync_copy(hlo_hbm.at[pl.ds(r0, R_BIG)],
                        h_st.at[pl.ds(r0, R_BIG)])

    @pl.when((sid < 15) & (cid == 1))
    def _():
        pltpu.sync_copy(hhi_hbm.at[pl.ds(r0, R_BIG)],
                        h_st.at[pl.ds(r0, R_BIG)])

    @pl.when((sid == 15) & (cid == 0))
    def _():
        pltpu.sync_copy(hlo_hbm.at[pl.ds(15 * R_BIG, R_SMALL)],
                        h_st.at[pl.ds(15 * R_BIG, R_SMALL)])

    @pl.when((sid == 15) & (cid == 1))
    def _():
        pltpu.sync_copy(hhi_hbm.at[pl.ds(15 * R_BIG, R_SMALL)],
                        h_st.at[pl.ds(15 * R_BIG, R_SMALL)])

    @pl.loop(0, n_z)
    def _(t):
        pltpu.sync_copy(rowss[0].at[pl.ds(0, 80)],
                        acc_sh.at[pl.ds(r0 + t * 80, 80)])
        pltpu.sync_copy(srows[0].at[pl.ds(0, 80)],
                        den_sh.at[pl.ds(r0 + t * 80, 80)])

    plsc.subcore_barrier()

    for t in range(PF):
        _issue(t * NS + sid, t)

    @pl.loop(0, LSTEPS // NSLOT)
    def _(u):
        for par in range(NSLOT):
            t = u * NSLOT + par
            _process(par)
            _issue((t + PF) * NS + sid, (par + PF) % NSLOT)

    for k in range(PF):
        _process((LSTEPS + k) % NSLOT)

    plsc.subcore_barrier()

    @pl.when(sid < 15)
    def _():
        pltpu.sync_copy(acc_sh.at[pl.ds(r0, R_BIG)],
                        accp_hbm.at[cid, pl.ds(r0, R_BIG)])
        pltpu.sync_copy(den_sh.at[pl.ds(r0, R_BIG)],
                        denp_hbm.at[cid, pl.ds(r0, R_BIG)])

    @pl.when(sid == 15)
    def _():
        pltpu.sync_copy(acc_sh.at[pl.ds(15 * R_BIG, R_SMALL)],
                        accp_hbm.at[cid, pl.ds(15 * R_BIG, R_SMALL)])
        pltpu.sync_copy(den_sh.at[pl.ds(15 * R_BIG, R_SMALL)],
                        denp_hbm.at[cid, pl.ds(15 * R_BIG, R_SMALL)])


def _sc_edge(src, dst, s_tab, d_tab, hlo, hhi):
    k = pl.kernel(
        _edge_body,
        out_type=[
            jax.ShapeDtypeStruct((NC, N, HH), _f32),
            jax.ShapeDtypeStruct((NC, N, 16), _f32),
        ],
        mesh=plsc.VectorSubcoreMesh(**_MESH),
        compiler_params=pltpu.CompilerParams(use_tc_tiling_on_sc=False),
        scratch_types=(
            [pltpu.VMEM((B,), jnp.int32)] * (2 * NSLOT)
            + [pltpu.VMEM((B, 16), _f32)] * (2 * NSLOT)
            + [pltpu.VMEM((B, HH), _f32)] * NSLOT
            + [pltpu.VMEM_SHARED((N, HH), _f32),
               pltpu.VMEM_SHARED((NP, HH), _f32),
               pltpu.VMEM_SHARED((NP, 16), _f32)]
            + [pltpu.SemaphoreType.DMA] * (3 * NSLOT)
        ),
    )
    return k(src, dst, s_tab, d_tab, hlo, hhi)


def _gather_body(x_hbm, wi_hbm, out_hbm, idx_v, rows_v, sem):
    wid = lax.axis_index("s") * NC + lax.axis_index("c")
    base = wid * VPW
    pltpu.sync_copy(wi_hbm.at[pl.ds(base, VPW)], idx_v)
    pltpu.async_copy(x_hbm.at[idx_v], rows_v, sem).wait()
    pltpu.sync_copy(rows_v, out_hbm.at[pl.ds(base, VPW)])


def _sc_gather(x, wi_pad):
    k = pl.kernel(
        _gather_body,
        out_type=jax.ShapeDtypeStruct((VPAD, HC), _f32),
        mesh=plsc.VectorSubcoreMesh(**_MESH),
        scratch_types=[
            pltpu.VMEM((VPW,), jnp.int32),
            pltpu.VMEM((VPW, HC), _f32),
            pltpu.SemaphoreType.DMA,
        ],
    )
    return k(x, wi_pad)


def kernel(embedding, to_children_edges, to_ancestor_edges, word_indexes, W,
           att_src, att_dst, bias):
    j = jnp.arange(HC)
    hp = jnp.arange(16)
    sel = (j[:, None] // OUT == hp[None, :] % HEADS).astype(_f32)
    asel = att_src.reshape(HC, 1) * sel
    adsel = att_dst.reshape(HC, 1) * sel
    b16 = ((hp[:, None] == j[None, :] // OUT) & (hp[:, None] < HEADS))
    b16 = b16.astype(_f32)

    pad_src = jnp.zeros((EPAD - E,), jnp.int32)
    pad_dst = jnp.full((EPAD - E,), N, jnp.int32)
    src1 = jnp.concatenate([to_children_edges[0], pad_src])
    dst1 = jnp.concatenate([to_children_edges[1], pad_dst])
    src2 = jnp.concatenate([to_ancestor_edges[0], pad_src])
    dst2 = jnp.concatenate([to_ancestor_edges[1], pad_dst])
    wi_pad = jnp.concatenate(
        [word_indexes, jnp.zeros((VPAD - V,), jnp.int32)])

    hlo1, hhi1, s1, d1 = _tc_proj(embedding, W, asel, adsel)
    accp1, denp1 = _sc_edge(src1, dst1, s1, jnp.pad(d1, ((0, 8), (0, 0))),
                            hlo1, hhi1)
    hlo2, hhi2, s2, d2 = _tc_norm_proj(accp1, denp1, b16, bias, W, asel,
                                       adsel)
    accp2, denp2 = _sc_edge(src2, dst2, s2, jnp.pad(d2, ((0, 8), (0, 0))),
                            hlo2, hhi2)
    x3 = _tc_norm(accp2, denp2, b16, bias)
    out = _sc_gather(x3, wi_pad)
    return out[:V]


# speedup vs baseline: 1.3240x; 1.3240x over previous
import jax
import jax.numpy as jnp
from jax import lax
from jax.experimental import pallas as pl
from jax.experimental.pallas import tpu as pltpu
from jax.experimental.pallas import tpu_sc as plsc

N = 10000
V = 8000
E = 320000
IN = 128
HEADS = 8
OUT = 16
HC = HEADS * OUT
HH = HC // 2
NEG = 0.2

NC, NS = 2, 16
NW = NC * NS
B = 128
CPT = 158
NCHUNK = CPT * NS
EPAD = NCHUNK * B
NP = N + 8

R_BIG = 640
R_SMALL = N - 15 * R_BIG
VPAD = 8192
VPW = VPAD // NW

_f32 = jnp.float32


def _proj(x, w, as_ref, ad_ref):
    h = lax.dot_general(x, w, (((1,), (1,)), ((), ())),
                        preferred_element_type=_f32)
    s = lax.dot_general(h, as_ref, (((1,), (0,)), ((), ())),
                        preferred_element_type=_f32)
    d = lax.dot_general(h, ad_ref, (((1,), (0,)), ((), ())),
                        preferred_element_type=_f32)
    return h, s, d


def _proj_body(x_ref, w_ref, as_ref, ad_ref, hlo_ref, hhi_ref, s_ref, d_ref):
    h, s, d = _proj(x_ref[...], w_ref[...], as_ref[...], ad_ref[...])
    hlo_ref[...] = h[:, :HH]
    hhi_ref[...] = h[:, HH:]
    s_ref[...] = s
    d_ref[...] = d


def _merge_x(acc_ref, den_ref, b16_ref, bias_ref):
    acc = jnp.concatenate([acc_ref[0], acc_ref[1]], axis=1)
    den = den_ref[0]
    denb = lax.dot_general(den, b16_ref[...], (((1,), (0,)), ((), ())),
                           preferred_element_type=_f32)
    return acc / (denb + 1e-16) + bias_ref[...]


def _norm_proj_body(acc_ref, den_ref, b16_ref, bias_ref, w_ref, as_ref,
                    ad_ref, hlo_ref, hhi_ref, s_ref, d_ref):
    x = _merge_x(acc_ref, den_ref, b16_ref, bias_ref)
    h, s, d = _proj(x, w_ref[...], as_ref[...], ad_ref[...])
    hlo_ref[...] = h[:, :HH]
    hhi_ref[...] = h[:, HH:]
    s_ref[...] = s
    d_ref[...] = d


def _norm_body(acc_ref, den_ref, b16_ref, bias_ref, x_ref):
    x_ref[...] = _merge_x(acc_ref, den_ref, b16_ref, bias_ref)


_R = 1000


def _tc_proj(x, w, asel, adsel):
    return pl.pallas_call(
        _proj_body,
        grid=(N // _R,),
        in_specs=[
            pl.BlockSpec((_R, IN), lambda i: (i, 0)),
            pl.BlockSpec((HC, IN), lambda i: (0, 0)),
            pl.BlockSpec((HC, 16), lambda i: (0, 0)),
            pl.BlockSpec((HC, 16), lambda i: (0, 0)),
        ],
        out_specs=[
            pl.BlockSpec((_R, HH), lambda i: (i, 0)),
            pl.BlockSpec((_R, HH), lambda i: (i, 0)),
            pl.BlockSpec((_R, 16), lambda i: (i, 0)),
            pl.BlockSpec((_R, 16), lambda i: (i, 0)),
        ],
        out_shape=[
            jax.ShapeDtypeStruct((N, HH), _f32),
            jax.ShapeDtypeStruct((N, HH), _f32),
            jax.ShapeDtypeStruct((N, 16), _f32),
            jax.ShapeDtypeStruct((N, 16), _f32),
        ],
    )(x, w, asel, adsel)


def _tc_norm_proj(accp, denp, b16, bias, w, asel, adsel):
    return pl.pallas_call(
        _norm_proj_body,
        grid=(N // _R,),
        in_specs=[
            pl.BlockSpec((NC, _R, HH), lambda i: (0, i, 0)),
            pl.BlockSpec((NC, _R, 16), lambda i: (0, i, 0)),
            pl.BlockSpec((16, HC), lambda i: (0, 0)),
            pl.BlockSpec((HC,), lambda i: (0,)),
            pl.BlockSpec((HC, IN), lambda i: (0, 0)),
            pl.BlockSpec((HC, 16), lambda i: (0, 0)),
            pl.BlockSpec((HC, 16), lambda i: (0, 0)),
        ],
        out_specs=[
            pl.BlockSpec((_R, HH), lambda i: (i, 0)),
            pl.BlockSpec((_R, HH), lambda i: (i, 0)),
            pl.BlockSpec((_R, 16), lambda i: (i, 0)),
            pl.BlockSpec((_R, 16), lambda i: (i, 0)),
        ],
        out_shape=[
            jax.ShapeDtypeStruct((N, HH), _f32),
            jax.ShapeDtypeStruct((N, HH), _f32),
            jax.ShapeDtypeStruct((N, 16), _f32),
            jax.ShapeDtypeStruct((N, 16), _f32),
        ],
    )(accp, denp, b16, bias, w, asel, adsel)


def _tc_norm(accp, denp, b16, bias):
    return pl.pallas_call(
        _norm_body,
        grid=(N // _R,),
        in_specs=[
            pl.BlockSpec((NC, _R, HH), lambda i: (0, i, 0)),
            pl.BlockSpec((NC, _R, 16), lambda i: (0, i, 0)),
            pl.BlockSpec((16, HC), lambda i: (0, 0)),
            pl.BlockSpec((HC,), lambda i: (0,)),
        ],
        out_specs=pl.BlockSpec((_R, HC), lambda i: (i, 0)),
        out_shape=jax.ShapeDtypeStruct((N, HC), _f32),
    )(accp, denp, b16, bias)


_MESH = dict(core_axis_name="c", subcore_axis_name="s", num_cores=NC,
             num_subcores=NS)

NSLOT = 3
PF = 2
LSTEPS = CPT - PF


def _edge_body(src_hbm, dst_hbm, s_hbm, d_hbm, hlo_hbm, hhi_hbm,
               accp_hbm, denp_hbm,
               sidx0, sidx1, sidx2,
               didx0, didx1, didx2,
               srow0, srow1, srow2,
               drow0, drow1, drow2,
               rows0, rows1, rows2,
               h_st, acc_sh, den_sh,
               g0, g1, g2, g3, g4, g5, g6, g7, g8,
               i0, i1, i2, e0, e1, e2):
    cid = lax.axis_index("c")
    sid = lax.axis_index("s")
    gsa = (g0, g1, g2)
    gsb = (g3, g4, g5)
    gsc = (g6, g7, g8)
    isems = (i0, i1, i2)
    ssems = (e0, e1, e2)
    sidxs = (sidx0, sidx1, sidx2)
    didxs = (didx0, didx1, didx2)
    srows = (srow0, srow1, srow2)
    drows = (drow0, drow1, drow2)
    rowss = (rows0, rows1, rows2)

    zero16 = jnp.zeros((16,), _f32)

    def _ids(c, par):
        base = c * B
        pltpu.async_copy(src_hbm.at[pl.ds(base, B)], sidxs[par], isems[par])
        pltpu.async_copy(dst_hbm.at[pl.ds(base, B)], didxs[par], isems[par])

    def _wait_ids(c, par):
        base = c * B
        pltpu.make_async_copy(src_hbm.at[pl.ds(base, B)], sidxs[par],
                              isems[par]).wait()
        pltpu.make_async_copy(dst_hbm.at[pl.ds(base, B)], didxs[par],
                              isems[par]).wait()

    def _gathers(par):
        pltpu.async_copy(s_hbm.at[sidxs[par]], srows[par], gsa[par])
        pltpu.async_copy(d_hbm.at[didxs[par]], drows[par], gsb[par])
        pltpu.async_copy(h_st.at[sidxs[par]], rowss[par], gsc[par])

    def _drain_scatters(par):
        pltpu.make_async_copy(srows[par], den_sh.at[didxs[par]],
                              ssems[par]).wait()
        pltpu.make_async_copy(rowss[par], acc_sh.at[didxs[par]],
                              ssems[par]).wait()

    def _process(par):
        srow, drow, rows = srows[par], drows[par], rowss[par]
        pltpu.make_async_copy(s_hbm.at[sidxs[par]], srow, gsa[par]).wait()
        pltpu.make_async_copy(d_hbm.at[didxs[par]], drow, gsb[par]).wait()
        pltpu.make_async_copy(h_st.at[sidxs[par]], rows, gsc[par]).wait()

        def _compute(hbase):
            @plsc.parallel_loop(0, B, unroll=2)
            def _(b):
                a = srow[b, :] + drow[b, :]
                a = jnp.where(a > 0, a, a * NEG)
                pv = jnp.exp(a)
                srow[b, :] = pv
                for j in range(HEADS // 2):
                    sc = jnp.full((16,), pv[hbase + j], _f32)
                    rows[b, pl.ds(16 * j, 16)] = (
                        rows[b, pl.ds(16 * j, 16)] * sc)

        @pl.when(cid == 0)
        def _():
            _compute(0)

        @pl.when(cid == 1)
        def _():
            _compute(HEADS // 2)

        pltpu.async_copy(srow, den_sh.at[didxs[par]], ssems[par], add=True)
        pltpu.async_copy(rows, acc_sh.at[didxs[par]], ssems[par], add=True)

    r0 = sid * R_BIG
    n_z = jnp.where(sid < 15, R_BIG // 80, R_SMALL // 80)

    @pl.loop(0, 80)
    def _(b):
        srows[0][b, :] = zero16
        for j in range(HEADS // 2):
            rowss[0][b, pl.ds(16 * j, 16)] = zero16

    @pl.when((sid < 15) & (cid == 0))
    def _():
        pltpu.sync_copy(hlo_hbm.at[pl.ds(r0, R_BIG)],
                        h_st.at[pl.ds(r0, R_BIG)])

    @pl.when((sid < 15) & (cid == 1))
    def _():
        pltpu.sync_copy(hhi_hbm.at[pl.ds(r0, R_BIG)],
                        h_st.at[pl.ds(r0, R_BIG)])

    @pl.when((sid == 15) & (cid == 0))
    def _():
        pltpu.sync_copy(hlo_hbm.at[pl.ds(15 * R_BIG, R_SMALL)],
                        h_st.at[pl.ds(15 * R_BIG, R_SMALL)])

    @pl.when((sid == 15) & (cid == 1))
    def _():
        pltpu.sync_copy(hhi_hbm.at[pl.ds(15 * R_BIG, R_SMALL)],
                        h_st.at[pl.ds(15 * R_BIG, R_SMALL)])

    @pl.loop(0, n_z)
    def _(t):
        pltpu.sync_copy(rowss[0].at[pl.ds(0, 80)],
                        acc_sh.at[pl.ds(r0 + t * 80, 80)])
        pltpu.sync_copy(srows[0].at[pl.ds(0, 80)],
                        den_sh.at[pl.ds(r0 + t * 80, 80)])

    plsc.subcore_barrier()

    def _c(t):
        return t * NS + sid

    _ids(_c(0), 0)
    _ids(_c(1), 1)
    _wait_ids(_c(0), 0)
    _gathers(0)

    @pl.loop(0, (CPT - 2) // NSLOT)
    def _(u):
        for par in range(NSLOT):
            t = u * NSLOT + par
            p1 = (par + 1) % NSLOT
            p2 = (par + 2) % NSLOT
            _process(par)
            _wait_ids(_c(t + 1), p1)
            _gathers(p1)

            @pl.when(t >= 1)
            def _(p2=p2):
                _drain_scatters(p2)

            _ids(_c(t + 2), p2)

    _process(0)
    _wait_ids(_c(CPT - 1), 1)
    _gathers(1)
    _drain_scatters(2)
    _process(1)
    _drain_scatters(0)
    _drain_scatters(1)

    plsc.subcore_barrier()

    @pl.when(sid < 15)
    def _():
        pltpu.sync_copy(acc_sh.at[pl.ds(r0, R_BIG)],
                        accp_hbm.at[cid, pl.ds(r0, R_BIG)])
        pltpu.sync_copy(den_sh.at[pl.ds(r0, R_BIG)],
                        denp_hbm.at[cid, pl.ds(r0, R_BIG)])

    @pl.when(sid == 15)
    def _():
        pltpu.sync_copy(acc_sh.at[pl.ds(15 * R_BIG, R_SMALL)],
                        accp_hbm.at[cid, pl.ds(15 * R_BIG, R_SMALL)])
        pltpu.sync_copy(den_sh.at[pl.ds(15 * R_BIG, R_SMALL)],
                        denp_hbm.at[cid, pl.ds(15 * R_BIG, R_SMALL)])


def _sc_edge(src, dst, s_tab, d_tab, hlo, hhi):
    k = pl.kernel(
        _edge_body,
        out_type=[
            jax.ShapeDtypeStruct((NC, N, HH), _f32),
            jax.ShapeDtypeStruct((NC, N, 16), _f32),
        ],
        mesh=plsc.VectorSubcoreMesh(**_MESH),
        compiler_params=pltpu.CompilerParams(use_tc_tiling_on_sc=False),
        scratch_types=(
            [pltpu.VMEM((B,), jnp.int32)] * (2 * NSLOT)
            + [pltpu.VMEM((B, 16), _f32)] * (2 * NSLOT)
            + [pltpu.VMEM((B, HH), _f32)] * NSLOT
            + [pltpu.VMEM_SHARED((N, HH), _f32),
               pltpu.VMEM_SHARED((NP, HH), _f32),
               pltpu.VMEM_SHARED((NP, 16), _f32)]
            + [pltpu.SemaphoreType.DMA] * (5 * NSLOT)
        ),
    )
    return k(src, dst, s_tab, d_tab, hlo, hhi)


def _gather_body(x_hbm, wi_hbm, out_hbm, idx_v, rows_v, sem):
    wid = lax.axis_index("s") * NC + lax.axis_index("c")
    base = wid * VPW
    pltpu.sync_copy(wi_hbm.at[pl.ds(base, VPW)], idx_v)
    pltpu.async_copy(x_hbm.at[idx_v], rows_v, sem).wait()
    pltpu.sync_copy(rows_v, out_hbm.at[pl.ds(base, VPW)])


def _sc_gather(x, wi_pad):
    k = pl.kernel(
        _gather_body,
        out_type=jax.ShapeDtypeStruct((VPAD, HC), _f32),
        mesh=plsc.VectorSubcoreMesh(**_MESH),
        scratch_types=[
            pltpu.VMEM((VPW,), jnp.int32),
            pltpu.VMEM((VPW, HC), _f32),
            pltpu.SemaphoreType.DMA,
        ],
    )
    return k(x, wi_pad)


def kernel(embedding, to_children_edges, to_ancestor_edges, word_indexes, W,
           att_src, att_dst, bias):
    j = jnp.arange(HC)
    hp = jnp.arange(16)
    sel = (j[:, None] // OUT == hp[None, :] % HEADS).astype(_f32)
    asel = att_src.reshape(HC, 1) * sel
    adsel = att_dst.reshape(HC, 1) * sel
    b16 = ((hp[:, None] == j[None, :] // OUT) & (hp[:, None] < HEADS))
    b16 = b16.astype(_f32)

    pad_src = jnp.zeros((EPAD - E,), jnp.int32)
    pad_dst = jnp.full((EPAD - E,), N, jnp.int32)
    src1 = jnp.concatenate([to_children_edges[0], pad_src])
    dst1 = jnp.concatenate([to_children_edges[1], pad_dst])
    src2 = jnp.concatenate([to_ancestor_edges[0], pad_src])
    dst2 = jnp.concatenate([to_ancestor_edges[1], pad_dst])
    wi_pad = jnp.concatenate(
        [word_indexes, jnp.zeros((VPAD - V,), jnp.int32)])

    hlo1, hhi1, s1, d1 = _tc_proj(embedding, W, asel, adsel)
    accp1, denp1 = _sc_edge(src1, dst1, s1, jnp.pad(d1, ((0, 8), (0, 0))),
                            hlo1, hhi1)
    hlo2, hhi2, s2, d2 = _tc_norm_proj(accp1, denp1, b16, bias, W, asel,
                                       adsel)
    accp2, denp2 = _sc_edge(src2, dst2, s2, jnp.pad(d2, ((0, 8), (0, 0))),
                            hlo2, hhi2)
    x3 = _tc_norm(accp2, denp2, b16, bias)
    out = _sc_gather(x3, wi_pad)
    return out[:V]


# speedup vs baseline: 1.3245x; 1.0004x over previous
import jax
import jax.numpy as jnp
from jax import lax
from jax.experimental import pallas as pl
from jax.experimental.pallas import tpu as pltpu
from jax.experimental.pallas import tpu_sc as plsc

N = 10000
V = 8000
E = 320000
IN = 128
HEADS = 8
OUT = 16
HC = HEADS * OUT
HH = HC // 2
NEG = 0.2

NC, NS = 2, 16
NW = NC * NS
B = 128
CPT = 158
NCHUNK = CPT * NS
EPAD = NCHUNK * B
NP = N + 8

R_BIG = 640
R_SMALL = N - 15 * R_BIG
VPAD = 8192
VPW = VPAD // NW

_f32 = jnp.float32


def _proj(x, w, as_ref, ad_ref):
    h = lax.dot_general(x, w, (((1,), (1,)), ((), ())),
                        preferred_element_type=_f32)
    s = lax.dot_general(h, as_ref, (((1,), (0,)), ((), ())),
                        preferred_element_type=_f32)
    d = lax.dot_general(h, ad_ref, (((1,), (0,)), ((), ())),
                        preferred_element_type=_f32)
    return h, s, d


def _proj_body(x_ref, w_ref, as_ref, ad_ref, hlo_ref, hhi_ref, s_ref, d_ref):
    h, s, d = _proj(x_ref[...], w_ref[...], as_ref[...], ad_ref[...])
    hlo_ref[...] = h[:, :HH]
    hhi_ref[...] = h[:, HH:]
    s_ref[...] = s
    d_ref[...] = d


def _merge_x(acc_ref, den_ref, b16_ref, bias_ref):
    acc = jnp.concatenate([acc_ref[0], acc_ref[1]], axis=1)
    den = den_ref[0]
    denb = lax.dot_general(den, b16_ref[...], (((1,), (0,)), ((), ())),
                           preferred_element_type=_f32)
    return acc / (denb + 1e-16) + bias_ref[...]


def _norm_proj_body(acc_ref, den_ref, b16_ref, bias_ref, w_ref, as_ref,
                    ad_ref, hlo_ref, hhi_ref, s_ref, d_ref):
    x = _merge_x(acc_ref, den_ref, b16_ref, bias_ref)
    h, s, d = _proj(x, w_ref[...], as_ref[...], ad_ref[...])
    hlo_ref[...] = h[:, :HH]
    hhi_ref[...] = h[:, HH:]
    s_ref[...] = s
    d_ref[...] = d


def _norm_body(acc_ref, den_ref, b16_ref, bias_ref, x_ref):
    x_ref[...] = _merge_x(acc_ref, den_ref, b16_ref, bias_ref)


_R = 1000


def _tc_proj(x, w, asel, adsel):
    return pl.pallas_call(
        _proj_body,
        grid=(N // _R,),
        in_specs=[
            pl.BlockSpec((_R, IN), lambda i: (i, 0)),
            pl.BlockSpec((HC, IN), lambda i: (0, 0)),
            pl.BlockSpec((HC, 16), lambda i: (0, 0)),
            pl.BlockSpec((HC, 16), lambda i: (0, 0)),
        ],
        out_specs=[
            pl.BlockSpec((_R, HH), lambda i: (i, 0)),
            pl.BlockSpec((_R, HH), lambda i: (i, 0)),
            pl.BlockSpec((_R, 16), lambda i: (i, 0)),
            pl.BlockSpec((_R, 16), lambda i: (i, 0)),
        ],
        out_shape=[
            jax.ShapeDtypeStruct((N, HH), _f32),
            jax.ShapeDtypeStruct((N, HH), _f32),
            jax.ShapeDtypeStruct((N, 16), _f32),
            jax.ShapeDtypeStruct((N, 16), _f32),
        ],
    )(x, w, asel, adsel)


def _tc_norm_proj(accp, denp, b16, bias, w, asel, adsel):
    return pl.pallas_call(
        _norm_proj_body,
        grid=(N // _R,),
        in_specs=[
            pl.BlockSpec((NC, _R, HH), lambda i: (0, i, 0)),
            pl.BlockSpec((NC, _R, 16), lambda i: (0, i, 0)),
            pl.BlockSpec((16, HC), lambda i: (0, 0)),
            pl.BlockSpec((HC,), lambda i: (0,)),
            pl.BlockSpec((HC, IN), lambda i: (0, 0)),
            pl.BlockSpec((HC, 16), lambda i: (0, 0)),
            pl.BlockSpec((HC, 16), lambda i: (0, 0)),
        ],
        out_specs=[
            pl.BlockSpec((_R, HH), lambda i: (i, 0)),
            pl.BlockSpec((_R, HH), lambda i: (i, 0)),
            pl.BlockSpec((_R, 16), lambda i: (i, 0)),
            pl.BlockSpec((_R, 16), lambda i: (i, 0)),
        ],
        out_shape=[
            jax.ShapeDtypeStruct((N, HH), _f32),
            jax.ShapeDtypeStruct((N, HH), _f32),
            jax.ShapeDtypeStruct((N, 16), _f32),
            jax.ShapeDtypeStruct((N, 16), _f32),
        ],
    )(accp, denp, b16, bias, w, asel, adsel)


def _tc_norm(accp, denp, b16, bias):
    return pl.pallas_call(
        _norm_body,
        grid=(N // _R,),
        in_specs=[
            pl.BlockSpec((NC, _R, HH), lambda i: (0, i, 0)),
            pl.BlockSpec((NC, _R, 16), lambda i: (0, i, 0)),
            pl.BlockSpec((16, HC), lambda i: (0, 0)),
            pl.BlockSpec((HC,), lambda i: (0,)),
        ],
        out_specs=pl.BlockSpec((_R, HC), lambda i: (i, 0)),
        out_shape=jax.ShapeDtypeStruct((N, HC), _f32),
    )(accp, denp, b16, bias)


_MESH = dict(core_axis_name="c", subcore_axis_name="s", num_cores=NC,
             num_subcores=NS)

NSLOT = 3
PF = 2
LSTEPS = CPT - PF


def _edge_body(src_hbm, dst_hbm, s_hbm, d_hbm, hlo_hbm, hhi_hbm,
               accp_hbm, denp_hbm,
               sidx0, sidx1, sidx2,
               didx0, didx1, didx2,
               srow0, srow1, srow2,
               drow0, drow1, drow2,
               rows0, rows1, rows2,
               h_st, acc_sh, den_sh,
               g0, g1, g2, g3, g4, g5, g6, g7, g8,
               i0, i1, i2, e0, e1, e2):
    cid = lax.axis_index("c")
    sid = lax.axis_index("s")
    gsa = (g0, g1, g2)
    gsb = (g3, g4, g5)
    gsc = (g6, g7, g8)
    isems = (i0, i1, i2)
    ssems = (e0, e1, e2)
    sidxs = (sidx0, sidx1, sidx2)
    didxs = (didx0, didx1, didx2)
    srows = (srow0, srow1, srow2)
    drows = (drow0, drow1, drow2)
    rowss = (rows0, rows1, rows2)

    zero16 = jnp.zeros((16,), _f32)

    def _ids(c, par):
        base = c * B
        pltpu.async_copy(src_hbm.at[pl.ds(base, B)], sidxs[par], isems[par])
        pltpu.async_copy(dst_hbm.at[pl.ds(base, B)], didxs[par], isems[par])

    def _wait_ids(c, par):
        base = c * B
        pltpu.make_async_copy(src_hbm.at[pl.ds(base, B)], sidxs[par],
                              isems[par]).wait()
        pltpu.make_async_copy(dst_hbm.at[pl.ds(base, B)], didxs[par],
                              isems[par]).wait()

    def _gathers(par):
        pltpu.async_copy(s_hbm.at[sidxs[par]], srows[par], gsa[par])
        pltpu.async_copy(d_hbm.at[didxs[par]], drows[par], gsb[par])
        pltpu.async_copy(h_st.at[sidxs[par]], rowss[par], gsc[par])

    def _drain_scatters(par):
        pltpu.make_async_copy(srows[par], den_sh.at[didxs[par]],
                              ssems[par]).wait()
        pltpu.make_async_copy(rowss[par], acc_sh.at[didxs[par]],
                              ssems[par]).wait()

    def _process(par):
        srow, drow, rows = srows[par], drows[par], rowss[par]
        pltpu.make_async_copy(s_hbm.at[sidxs[par]], srow, gsa[par]).wait()
        pltpu.make_async_copy(d_hbm.at[didxs[par]], drow, gsb[par]).wait()
        pltpu.make_async_copy(h_st.at[sidxs[par]], rows, gsc[par]).wait()

        def _compute(hbase):
            @plsc.parallel_loop(0, B, unroll=4)
            def _(b):
                a = srow[b, :] + drow[b, :]
                a = jnp.where(a > 0, a, a * NEG)
                pv = jnp.exp(a)
                srow[b, :] = pv
                for j in range(HEADS // 2):
                    sc = jnp.full((16,), pv[hbase + j], _f32)
                    rows[b, pl.ds(16 * j, 16)] = (
                        rows[b, pl.ds(16 * j, 16)] * sc)

        @pl.when(cid == 0)
        def _():
            _compute(0)

        @pl.when(cid == 1)
        def _():
            _compute(HEADS // 2)

        pltpu.async_copy(srow, den_sh.at[didxs[par]], ssems[par], add=True)
        pltpu.async_copy(rows, acc_sh.at[didxs[par]], ssems[par], add=True)

    r0 = sid * R_BIG
    n_z = jnp.where(sid < 15, R_BIG // 80, R_SMALL // 80)

    @pl.loop(0, 80)
    def _(b):
        srows[0][b, :] = zero16
        for j in range(HEADS // 2):
            rowss[0][b, pl.ds(16 * j, 16)] = zero16

    @pl.when((sid < 15) & (cid == 0))
    def _():
        pltpu.sync_copy(hlo_hbm.at[pl.ds(r0, R_BIG)],
                        h_st.at[pl.ds(r0, R_BIG)])

    @pl.when((sid < 15) & (cid == 1))
    def _():
        pltpu.sync_copy(hhi_hbm.at[pl.ds(r0, R_BIG)],
                        h_st.at[pl.ds(r0, R_BIG)])

    @pl.when((sid == 15) & (cid == 0))
    def _():
        pltpu.sync_copy(hlo_hbm.at[pl.ds(15 * R_BIG, R_SMALL)],
                        h_st.at[pl.ds(15 * R_BIG, R_SMALL)])

    @pl.when((sid == 15) & (cid == 1))
    def _():
        pltpu.sync_copy(hhi_hbm.at[pl.ds(15 * R_BIG, R_SMALL)],
                        h_st.at[pl.ds(15 * R_BIG, R_SMALL)])

    @pl.loop(0, n_z)
    def _(t):
        pltpu.sync_copy(rowss[0].at[pl.ds(0, 80)],
                        acc_sh.at[pl.ds(r0 + t * 80, 80)])
        pltpu.sync_copy(srows[0].at[pl.ds(0, 80)],
                        den_sh.at[pl.ds(r0 + t * 80, 80)])

    plsc.subcore_barrier()

    def _c(t):
        return t * NS + sid

    _ids(_c(0), 0)
    _ids(_c(1), 1)
    _wait_ids(_c(0), 0)
    _gathers(0)

    @pl.loop(0, (CPT - 2) // NSLOT)
    def _(u):
        for par in range(NSLOT):
            t = u * NSLOT + par
            p1 = (par + 1) % NSLOT
            p2 = (par + 2) % NSLOT
            _process(par)
            _wait_ids(_c(t + 1), p1)
            _gathers(p1)

            @pl.when(t >= 1)
            def _(p2=p2):
                _drain_scatters(p2)

            _ids(_c(t + 2), p2)

    _process(0)
    _wait_ids(_c(CPT - 1), 1)
    _gathers(1)
    _drain_scatters(2)
    _process(1)
    _drain_scatters(0)
    _drain_scatters(1)

    plsc.subcore_barrier()

    @pl.when(sid < 15)
    def _():
        pltpu.sync_copy(acc_sh.at[pl.ds(r0, R_BIG)],
                        accp_hbm.at[cid, pl.ds(r0, R_BIG)])
        pltpu.sync_copy(den_sh.at[pl.ds(r0, R_BIG)],
                        denp_hbm.at[cid, pl.ds(r0, R_BIG)])

    @pl.when(sid == 15)
    def _():
        pltpu.sync_copy(acc_sh.at[pl.ds(15 * R_BIG, R_SMALL)],
                        accp_hbm.at[cid, pl.ds(15 * R_BIG, R_SMALL)])
        pltpu.sync_copy(den_sh.at[pl.ds(15 * R_BIG, R_SMALL)],
                        denp_hbm.at[cid, pl.ds(15 * R_BIG, R_SMALL)])


def _sc_edge(src, dst, s_tab, d_tab, hlo, hhi):
    k = pl.kernel(
        _edge_body,
        out_type=[
            jax.ShapeDtypeStruct((NC, N, HH), _f32),
            jax.ShapeDtypeStruct((NC, N, 16), _f32),
        ],
        mesh=plsc.VectorSubcoreMesh(**_MESH),
        compiler_params=pltpu.CompilerParams(use_tc_tiling_on_sc=False),
        scratch_types=(
            [pltpu.VMEM((B,), jnp.int32)] * (2 * NSLOT)
            + [pltpu.VMEM((B, 16), _f32)] * (2 * NSLOT)
            + [pltpu.VMEM((B, HH), _f32)] * NSLOT
            + [pltpu.VMEM_SHARED((N, HH), _f32),
               pltpu.VMEM_SHARED((NP, HH), _f32),
               pltpu.VMEM_SHARED((NP, 16), _f32)]
            + [pltpu.SemaphoreType.DMA] * (5 * NSLOT)
        ),
    )
    return k(src, dst, s_tab, d_tab, hlo, hhi)


def _gather_body(x_hbm, wi_hbm, out_hbm, idx_v, rows_v, sem):
    wid = lax.axis_index("s") * NC + lax.axis_index("c")
    base = wid * VPW
    pltpu.sync_copy(wi_hbm.at[pl.ds(base, VPW)], idx_v)
    pltpu.async_copy(x_hbm.at[idx_v], rows_v, sem).wait()
    pltpu.sync_copy(rows_v, out_hbm.at[pl.ds(base, VPW)])


def _sc_gather(x, wi_pad):
    k = pl.kernel(
        _gather_body,
        out_type=jax.ShapeDtypeStruct((VPAD, HC), _f32),
        mesh=plsc.VectorSubcoreMesh(**_MESH),
        scratch_types=[
            pltpu.VMEM((VPW,), jnp.int32),
            pltpu.VMEM((VPW, HC), _f32),
            pltpu.SemaphoreType.DMA,
        ],
    )
    return k(x, wi_pad)


def kernel(embedding, to_children_edges, to_ancestor_edges, word_indexes, W,
           att_src, att_dst, bias):
    j = jnp.arange(HC)
    hp = jnp.arange(16)
    sel = (j[:, None] // OUT == hp[None, :] % HEADS).astype(_f32)
    asel = att_src.reshape(HC, 1) * sel
    adsel = att_dst.reshape(HC, 1) * sel
    b16 = ((hp[:, None] == j[None, :] // OUT) & (hp[:, None] < HEADS))
    b16 = b16.astype(_f32)

    pad_src = jnp.zeros((EPAD - E,), jnp.int32)
    pad_dst = jnp.full((EPAD - E,), N, jnp.int32)
    src1 = jnp.concatenate([to_children_edges[0], pad_src])
    dst1 = jnp.concatenate([to_children_edges[1], pad_dst])
    src2 = jnp.concatenate([to_ancestor_edges[0], pad_src])
    dst2 = jnp.concatenate([to_ancestor_edges[1], pad_dst])
    wi_pad = jnp.concatenate(
        [word_indexes, jnp.zeros((VPAD - V,), jnp.int32)])

    hlo1, hhi1, s1, d1 = _tc_proj(embedding, W, asel, adsel)
    accp1, denp1 = _sc_edge(src1, dst1, s1, jnp.pad(d1, ((0, 8), (0, 0))),
                            hlo1, hhi1)
    hlo2, hhi2, s2, d2 = _tc_norm_proj(accp1, denp1, b16, bias, W, asel,
                                       adsel)
    accp2, denp2 = _sc_edge(src2, dst2, s2, jnp.pad(d2, ((0, 8), (0, 0))),
                            hlo2, hhi2)
    x3 = _tc_norm(accp2, denp2, b16, bias)
    out = _sc_gather(x3, wi_pad)
    return out[:V]


# speedup vs baseline: 1.3802x; 1.0421x over previous
import jax
import jax.numpy as jnp
from jax import lax
from jax.experimental import pallas as pl
from jax.experimental.pallas import tpu as pltpu
from jax.experimental.pallas import tpu_sc as plsc

N = 10000
V = 8000
E = 320000
IN = 128
HEADS = 8
OUT = 16
HC = HEADS * OUT
HH = HC // 2
NEG = 0.2

NC, NS = 2, 16
NW = NC * NS
B = 128
CPT = 158
NCHUNK = CPT * NS
EPAD = NCHUNK * B
NP = N + 8

R_BIG = 640
R_SMALL = N - 15 * R_BIG
VPAD = 8192
VPW = VPAD // NW

_f32 = jnp.float32


def _proj(x, w, as_ref, ad_ref):
    h = lax.dot_general(x, w, (((1,), (1,)), ((), ())),
                        preferred_element_type=_f32)
    s = lax.dot_general(h, as_ref, (((1,), (0,)), ((), ())),
                        preferred_element_type=_f32)
    d = lax.dot_general(h, ad_ref, (((1,), (0,)), ((), ())),
                        preferred_element_type=_f32)
    return h, s, d


def _proj_body(x_ref, w_ref, as_ref, ad_ref, hlo_ref, hhi_ref, s_ref, d_ref):
    h, s, d = _proj(x_ref[...], w_ref[...], as_ref[...], ad_ref[...])
    hlo_ref[...] = h[:, :HH]
    hhi_ref[...] = h[:, HH:]
    s_ref[...] = s
    d_ref[...] = d


def _merge_x(acc_ref, den_ref, b16_ref, bias_ref):
    acc = jnp.concatenate([acc_ref[0], acc_ref[1]], axis=1)
    den = den_ref[0]
    denb = lax.dot_general(den, b16_ref[...], (((1,), (0,)), ((), ())),
                           preferred_element_type=_f32)
    return acc / (denb + 1e-16) + bias_ref[...]


def _norm_proj_body(acc_ref, den_ref, b16_ref, bias_ref, w_ref, as_ref,
                    ad_ref, hlo_ref, hhi_ref, s_ref, d_ref):
    x = _merge_x(acc_ref, den_ref, b16_ref, bias_ref)
    h, s, d = _proj(x, w_ref[...], as_ref[...], ad_ref[...])
    hlo_ref[...] = h[:, :HH]
    hhi_ref[...] = h[:, HH:]
    s_ref[...] = s
    d_ref[...] = d


def _norm_body(acc_ref, den_ref, b16_ref, bias_ref, x_ref):
    x_ref[...] = _merge_x(acc_ref, den_ref, b16_ref, bias_ref)


_R = 1000


def _tc_proj(x, w, asel, adsel):
    return pl.pallas_call(
        _proj_body,
        grid=(N // _R,),
        in_specs=[
            pl.BlockSpec((_R, IN), lambda i: (i, 0)),
            pl.BlockSpec((HC, IN), lambda i: (0, 0)),
            pl.BlockSpec((HC, 16), lambda i: (0, 0)),
            pl.BlockSpec((HC, 16), lambda i: (0, 0)),
        ],
        out_specs=[
            pl.BlockSpec((_R, HH), lambda i: (i, 0)),
            pl.BlockSpec((_R, HH), lambda i: (i, 0)),
            pl.BlockSpec((_R, 16), lambda i: (i, 0)),
            pl.BlockSpec((_R, 16), lambda i: (i, 0)),
        ],
        out_shape=[
            jax.ShapeDtypeStruct((N, HH), _f32),
            jax.ShapeDtypeStruct((N, HH), _f32),
            jax.ShapeDtypeStruct((N, 16), _f32),
            jax.ShapeDtypeStruct((N, 16), _f32),
        ],
    )(x, w, asel, adsel)


def _tc_norm_proj(accp, denp, b16, bias, w, asel, adsel):
    return pl.pallas_call(
        _norm_proj_body,
        grid=(N // _R,),
        in_specs=[
            pl.BlockSpec((NC, _R, HH), lambda i: (0, i, 0)),
            pl.BlockSpec((NC, _R, 16), lambda i: (0, i, 0)),
            pl.BlockSpec((16, HC), lambda i: (0, 0)),
            pl.BlockSpec((HC,), lambda i: (0,)),
            pl.BlockSpec((HC, IN), lambda i: (0, 0)),
            pl.BlockSpec((HC, 16), lambda i: (0, 0)),
            pl.BlockSpec((HC, 16), lambda i: (0, 0)),
        ],
        out_specs=[
            pl.BlockSpec((_R, HH), lambda i: (i, 0)),
            pl.BlockSpec((_R, HH), lambda i: (i, 0)),
            pl.BlockSpec((_R, 16), lambda i: (i, 0)),
            pl.BlockSpec((_R, 16), lambda i: (i, 0)),
        ],
        out_shape=[
            jax.ShapeDtypeStruct((N, HH), _f32),
            jax.ShapeDtypeStruct((N, HH), _f32),
            jax.ShapeDtypeStruct((N, 16), _f32),
            jax.ShapeDtypeStruct((N, 16), _f32),
        ],
    )(accp, denp, b16, bias, w, asel, adsel)


def _tc_norm(accp, denp, b16, bias):
    return pl.pallas_call(
        _norm_body,
        grid=(N // _R,),
        in_specs=[
            pl.BlockSpec((NC, _R, HH), lambda i: (0, i, 0)),
            pl.BlockSpec((NC, _R, 16), lambda i: (0, i, 0)),
            pl.BlockSpec((16, HC), lambda i: (0, 0)),
            pl.BlockSpec((HC,), lambda i: (0,)),
        ],
        out_specs=pl.BlockSpec((_R, HC), lambda i: (i, 0)),
        out_shape=jax.ShapeDtypeStruct((N, HC), _f32),
    )(accp, denp, b16, bias)


_MESH = dict(core_axis_name="c", subcore_axis_name="s", num_cores=NC,
             num_subcores=NS)

NSLOT = 3
PF = 2
LSTEPS = CPT - PF


def _edge_body(src_hbm, dst_hbm, s_hbm, d_hbm, hlo_hbm, hhi_hbm,
               accp_hbm, denp_hbm,
               sidx0, sidx1, sidx2,
               didx0, didx1, didx2,
               srow0, srow1, srow2,
               drow0, drow1, drow2,
               rows0, rows1, rows2,
               h_st, acc_sh, den_sh,
               g0, g1, g2, g3, g4, g5, g6, g7, g8,
               i0, i1, i2, e0, e1, e2):
    cid = lax.axis_index("c")
    sid = lax.axis_index("s")
    gsa = (g0, g1, g2)
    gsb = (g3, g4, g5)
    gsc = (g6, g7, g8)
    isems = (i0, i1, i2)
    ssems = (e0, e1, e2)
    sidxs = (sidx0, sidx1, sidx2)
    didxs = (didx0, didx1, didx2)
    srows = (srow0, srow1, srow2)
    drows = (drow0, drow1, drow2)
    rowss = (rows0, rows1, rows2)

    zero16 = jnp.zeros((16,), _f32)

    def _ids(c, par):
        base = c * B
        pltpu.async_copy(src_hbm.at[pl.ds(base, B)], sidxs[par], isems[par])
        pltpu.async_copy(dst_hbm.at[pl.ds(base, B)], didxs[par], isems[par])

    def _wait_ids(c, par):
        base = c * B
        pltpu.make_async_copy(src_hbm.at[pl.ds(base, B)], sidxs[par],
                              isems[par]).wait()
        pltpu.make_async_copy(dst_hbm.at[pl.ds(base, B)], didxs[par],
                              isems[par]).wait()

    def _gathers(par):
        pltpu.async_copy(s_hbm.at[sidxs[par]], srows[par], gsa[par])
        pltpu.async_copy(d_hbm.at[didxs[par]], drows[par], gsb[par])
        pltpu.async_copy(h_st.at[sidxs[par]], rowss[par], gsc[par])

    def _drain_scatters(par):
        pltpu.make_async_copy(srows[par], den_sh.at[didxs[par]],
                              ssems[par]).wait()
        pltpu.make_async_copy(rowss[par], acc_sh.at[didxs[par]],
                              ssems[par]).wait()

    def _process(par):
        srow, drow, rows = srows[par], drows[par], rowss[par]
        pltpu.make_async_copy(s_hbm.at[sidxs[par]], srow, gsa[par]).wait()
        pltpu.make_async_copy(d_hbm.at[didxs[par]], drow, gsb[par]).wait()
        pltpu.make_async_copy(h_st.at[sidxs[par]], rows, gsc[par]).wait()

        def _compute(hbase):
            @plsc.parallel_loop(0, B, unroll=4)
            def _(b):
                a = srow[b, :] + drow[b, :]
                a = jnp.where(a > 0, a, a * NEG)
                pv = jnp.exp(a)
                srow[b, :] = pv
                for j in range(HEADS // 2):
                    sc = jnp.full((16,), pv[hbase + j], _f32)
                    rows[b, pl.ds(16 * j, 16)] = (
                        rows[b, pl.ds(16 * j, 16)] * sc)

        @pl.when(cid == 0)
        def _():
            _compute(0)

        @pl.when(cid == 1)
        def _():
            _compute(HEADS // 2)

        pltpu.async_copy(srow, den_sh.at[didxs[par]], ssems[par], add=True)
        pltpu.async_copy(rows, acc_sh.at[didxs[par]], ssems[par], add=True)

    r0 = sid * R_BIG
    n_z = jnp.where(sid < 15, R_BIG // 80, R_SMALL // 80)

    @pl.loop(0, 80)
    def _(b):
        srows[0][b, :] = zero16
        for j in range(HEADS // 2):
            rowss[0][b, pl.ds(16 * j, 16)] = zero16

    @pl.when((sid < 15) & (cid == 0))
    def _():
        pltpu.sync_copy(hlo_hbm.at[pl.ds(r0, R_BIG)],
                        h_st.at[pl.ds(r0, R_BIG)])

    @pl.when((sid < 15) & (cid == 1))
    def _():
        pltpu.sync_copy(hhi_hbm.at[pl.ds(r0, R_BIG)],
                        h_st.at[pl.ds(r0, R_BIG)])

    @pl.when((sid == 15) & (cid == 0))
    def _():
        pltpu.sync_copy(hlo_hbm.at[pl.ds(15 * R_BIG, R_SMALL)],
                        h_st.at[pl.ds(15 * R_BIG, R_SMALL)])

    @pl.when((sid == 15) & (cid == 1))
    def _():
        pltpu.sync_copy(hhi_hbm.at[pl.ds(15 * R_BIG, R_SMALL)],
                        h_st.at[pl.ds(15 * R_BIG, R_SMALL)])

    @pl.loop(0, n_z)
    def _(t):
        pltpu.sync_copy(rowss[0].at[pl.ds(0, 80)],
                        acc_sh.at[pl.ds(r0 + t * 80, 80)])
        pltpu.sync_copy(srows[0].at[pl.ds(0, 80)],
                        den_sh.at[pl.ds(r0 + t * 80, 80)])

    plsc.subcore_barrier()

    def _c(t):
        return t * NS + sid

    _ids(_c(0), 0)
    _ids(_c(1), 1)
    _wait_ids(_c(0), 0)
    _gathers(0)

    @pl.loop(0, (CPT - 2) // NSLOT)
    def _(u):
        for par in range(NSLOT):
            t = u * NSLOT + par
            p1 = (par + 1) % NSLOT
            p2 = (par + 2) % NSLOT
            _process(par)
            _wait_ids(_c(t + 1), p1)
            _gathers(p1)

            @pl.when(t >= 1)
            def _(p2=p2):
                _drain_scatters(p2)

            _ids(_c(t + 2), p2)

    _process(0)
    _wait_ids(_c(CPT - 1), 1)
    _gathers(1)
    _drain_scatters(2)
    _process(1)
    _drain_scatters(0)
    _drain_scatters(1)

    plsc.subcore_barrier()

    @pl.when(sid < 15)
    def _():
        pltpu.sync_copy(acc_sh.at[pl.ds(r0, R_BIG)],
                        accp_hbm.at[cid, pl.ds(r0, R_BIG)])
        pltpu.sync_copy(den_sh.at[pl.ds(r0, R_BIG)],
                        denp_hbm.at[cid, pl.ds(r0, R_BIG)])

    @pl.when(sid == 15)
    def _():
        pltpu.sync_copy(acc_sh.at[pl.ds(15 * R_BIG, R_SMALL)],
                        accp_hbm.at[cid, pl.ds(15 * R_BIG, R_SMALL)])
        pltpu.sync_copy(den_sh.at[pl.ds(15 * R_BIG, R_SMALL)],
                        denp_hbm.at[cid, pl.ds(15 * R_BIG, R_SMALL)])


def _sc_edge(src, dst, s_tab, d_tab, hlo, hhi):
    k = pl.kernel(
        _edge_body,
        out_type=[
            jax.ShapeDtypeStruct((NC, N, HH), _f32),
            jax.ShapeDtypeStruct((NC, N, 16), _f32),
        ],
        mesh=plsc.VectorSubcoreMesh(**_MESH),
        compiler_params=pltpu.CompilerParams(use_tc_tiling_on_sc=False),
        scratch_types=(
            [pltpu.VMEM((B,), jnp.int32)] * (2 * NSLOT)
            + [pltpu.VMEM((B, 16), _f32)] * (2 * NSLOT)
            + [pltpu.VMEM((B, HH), _f32)] * NSLOT
            + [pltpu.VMEM_SHARED((N, HH), _f32),
               pltpu.VMEM_SHARED((NP, HH), _f32),
               pltpu.VMEM_SHARED((NP, 16), _f32)]
            + [pltpu.SemaphoreType.DMA] * (5 * NSLOT)
        ),
    )
    return k(src, dst, s_tab, d_tab, hlo, hhi)


def _gather_body(acc_hbm, den_hbm, bias_hbm, wi_hbm, out_hbm,
                 idx_v, a0_v, a1_v, dn_v, out_v, bias_v, s0, s1, s2, s3):
    wid = lax.axis_index("s") * NC + lax.axis_index("c")
    base = wid * VPW
    pltpu.sync_copy(bias_hbm, bias_v)
    pltpu.sync_copy(wi_hbm.at[pl.ds(base, VPW)], idx_v)
    pltpu.async_copy(acc_hbm.at[0].at[idx_v], a0_v, s0)
    pltpu.async_copy(acc_hbm.at[1].at[idx_v], a1_v, s1)
    pltpu.async_copy(den_hbm.at[0].at[idx_v], dn_v, s2)
    bias_r = [bias_v[pl.ds(16 * g, 16)] for g in range(HEADS)]
    pltpu.make_async_copy(acc_hbm.at[0].at[idx_v], a0_v, s0).wait()
    pltpu.make_async_copy(acc_hbm.at[1].at[idx_v], a1_v, s1).wait()
    pltpu.make_async_copy(den_hbm.at[0].at[idx_v], dn_v, s2).wait()

    @plsc.parallel_loop(0, VPW, unroll=2)
    def _(b):
        rv = 1.0 / (dn_v[b, :] + 1e-16)
        for g in range(HEADS // 2):
            sc = jnp.full((16,), rv[g], _f32)
            out_v[b, pl.ds(16 * g, 16)] = (
                a0_v[b, pl.ds(16 * g, 16)] * sc + bias_r[g])
        for g in range(HEADS // 2, HEADS):
            sc = jnp.full((16,), rv[g], _f32)
            out_v[b, pl.ds(16 * g, 16)] = (
                a1_v[b, pl.ds(16 * (g - 4), 16)] * sc + bias_r[g])

    pltpu.sync_copy(out_v, out_hbm.at[pl.ds(base, VPW)])


def _sc_gather(accp, denp, bias, wi_pad):
    k = pl.kernel(
        _gather_body,
        out_type=jax.ShapeDtypeStruct((VPAD, HC), _f32),
        mesh=plsc.VectorSubcoreMesh(**_MESH),
        compiler_params=pltpu.CompilerParams(use_tc_tiling_on_sc=False),
        scratch_types=[
            pltpu.VMEM((VPW,), jnp.int32),
            pltpu.VMEM((VPW, HH), _f32),
            pltpu.VMEM((VPW, HH), _f32),
            pltpu.VMEM((VPW, 16), _f32),
            pltpu.VMEM((VPW, HC), _f32),
            pltpu.VMEM((HC,), _f32),
            pltpu.SemaphoreType.DMA,
            pltpu.SemaphoreType.DMA,
            pltpu.SemaphoreType.DMA,
            pltpu.SemaphoreType.DMA,
        ],
    )
    return k(accp, denp, bias, wi_pad)


def kernel(embedding, to_children_edges, to_ancestor_edges, word_indexes, W,
           att_src, att_dst, bias):
    j = jnp.arange(HC)
    hp = jnp.arange(16)
    sel = (j[:, None] // OUT == hp[None, :] % HEADS).astype(_f32)
    asel = att_src.reshape(HC, 1) * sel
    adsel = att_dst.reshape(HC, 1) * sel
    b16 = ((hp[:, None] == j[None, :] // OUT) & (hp[:, None] < HEADS))
    b16 = b16.astype(_f32)

    pad_src = jnp.zeros((EPAD - E,), jnp.int32)
    pad_dst = jnp.full((EPAD - E,), N, jnp.int32)
    src1 = jnp.concatenate([to_children_edges[0], pad_src])
    dst1 = jnp.concatenate([to_children_edges[1], pad_dst])
    src2 = jnp.concatenate([to_ancestor_edges[0], pad_src])
    dst2 = jnp.concatenate([to_ancestor_edges[1], pad_dst])
    wi_pad = jnp.concatenate(
        [word_indexes, jnp.zeros((VPAD - V,), jnp.int32)])

    hlo1, hhi1, s1, d1 = _tc_proj(embedding, W, asel, adsel)
    accp1, denp1 = _sc_edge(src1, dst1, s1, jnp.pad(d1, ((0, 8), (0, 0))),
                            hlo1, hhi1)
    hlo2, hhi2, s2, d2 = _tc_norm_proj(accp1, denp1, b16, bias, W, asel,
                                       adsel)
    accp2, denp2 = _sc_edge(src2, dst2, s2, jnp.pad(d2, ((0, 8), (0, 0))),
                            hlo2, hhi2)
    out = _sc_gather(accp2, denp2, bias, wi_pad)
    return out[:V]
